# baseline (speedup 1.0000x reference)
import jax
import jax.numpy as jnp
from jax import lax
from jax.experimental import pallas as pl
from jax.experimental.pallas import tpu as pltpu
from jax.experimental.pallas import tpu_sc as plsc

B = 4096
SLOT = 26
EMB = 64
VOCAB = 1000000
NNZ = B * SLOT

NC = 2
NS = 16
NW = NC * NS
WSPAN = 31360
PADCOLS = 1000064
CHUNK_COLS = 1024
IW = 2048
NWIN = NNZ // IW
HCAP = 16384
CSEG = 4096
ROWCAP = 128
OUT_ROWS = NNZ + ROWCAP

_i32 = jnp.int32


def _sweep_body(tab_t, idx_hbm, out_hbm, idxwin, hitv, hitp, chv, chp,
                stage, rows, posb, dsem, osem):
    wid = lax.axis_index("s") * NC + lax.axis_index("c")
    lo = wid * WSPAN
    hi = jnp.minimum(lo + WSPAN, VOCAB)
    nchunk = (hi - lo + CHUNK_COLS - 1) // CHUNK_COLS
    iota = lax.iota(_i32, 16)
    sink = jnp.full((16,), NNZ, _i32)

    def _reset_posb():
        for g in range(ROWCAP // 16):
            posb[pl.ds(g * 16, 16)] = sink

    def _flush():
        pltpu.async_copy(rows, out_hbm.at[posb], osem).wait()
        _reset_posb()

    _reset_posb()

    def _pass_cond(st):
        p, tot = st
        return jnp.logical_or(p == 0, p * HCAP < tot)

    def _pass_body(st):
        p, _ = st
        done = p * HCAP

        def _filter_window(w, c):
            tot, hs = c
            pltpu.sync_copy(idx_hbm.at[pl.ds(w * IW, IW)], idxwin)

            def _grp(g, c2):
                tot2, hs2 = c2
                v = idxwin[pl.ds(g * 16, 16)]
                m = jnp.logical_and(v >= lo, v < hi)
                mi = m.astype(_i32)
                rank = tot2 + plsc.cumsum(mi)
                keep = jnp.logical_and(
                    m, jnp.logical_and(rank > done, rank <= done + HCAP))
                plsc.store_compressed(hitv.at[pl.ds(hs2, 16)], v, mask=keep)
                plsc.store_compressed(
                    hitp.at[pl.ds(hs2, 16)], w * IW + g * 16 + iota, mask=keep)
                return (tot2 + jnp.sum(mi),
                        hs2 + jnp.sum(keep.astype(_i32)))

            return pl.loop(0, IW // 16, init_carry=c, unroll=8)(_grp)

        tot, hs = pl.loop(
            0, NWIN, init_carry=(_i32(0), _i32(0)))(_filter_window)
        ngrp = (hs + 15) // 16
        nseg = jnp.maximum(_i32(1), (hs + CSEG - 1) // CSEG)

        @pl.loop(0, nchunk, init_carry=(_i32(0),))
        def _chunk(c, cc):
            (nrows0,) = cc
            clo = lo + c * CHUNK_COLS
            chi = jnp.minimum(clo + CHUNK_COLS, hi)
            col0 = jnp.minimum(clo, PADCOLS - CHUNK_COLS)
            for tf in range(8):
                pltpu.async_copy(
                    tab_t.at[pl.ds(tf * 8, 8), pl.ds(col0, CHUNK_COLS)],
                    stage.at[tf], dsem)
            for tf in range(8):
                pltpu.make_async_copy(
                    tab_t.at[pl.ds(tf * 8, 8), pl.ds(col0, CHUNK_COLS)],
                    stage.at[tf], dsem).wait()

            def _segment(s, sc_carry):
                (nrows_in,) = sc_carry

                def _rescan(g, c3):
                    rc, sc = c3
                    for k in range(4):
                        off = g * 64 + k * 16
                        v = hitv[pl.ds(off, 16)]
                        pz = hitp[pl.ds(off, 16)]
                        gv = jnp.logical_and(
                            iota < hs - off,
                            jnp.logical_and(v >= clo, v < chi))
                        gi = gv.astype(_i32)
                        rank = rc + plsc.cumsum(gi)
                        keep = jnp.logical_and(
                            gv, jnp.logical_and(rank > s * CSEG,
                                                rank <= s * CSEG + CSEG))
                        plsc.store_compressed(
                            chv.at[pl.ds(sc, 16)], v, mask=keep)
                        plsc.store_compressed(
                            chp.at[pl.ds(sc, 16)], pz, mask=keep)
                        rc = rc + jnp.sum(gi)
                        sc = sc + jnp.sum(keep.astype(_i32))
                    return (rc, sc)

                _, sc = pl.loop(
                    0, (hs + 63) // 64,
                    init_carry=(_i32(0), _i32(0)))(_rescan)

                def _extract(g, c4):
                    (nr_in,) = c4

                    @pl.when(nr_in > ROWCAP - 16)
                    def _():
                        _flush()

                    nr = jnp.where(nr_in > ROWCAP - 16, _i32(0), nr_in)
                    v = chv[pl.ds(g * 16, 16)]
                    pz = chp[pl.ds(g * 16, 16)]
                    valid = iota < sc - g * 16
                    vrel = v - col0
                    slots = nr + iota
                    for e in range(EMB):
                        colv = plsc.load_gather(
                            stage,
                            [jnp.full((16,), e // 8, _i32),
                             jnp.full((16,), e % 8, _i32),
                             vrel], mask=valid)
                        plsc.store_scatter(
                            rows, [slots, jnp.full((16,), e, _i32)],
                            colv, mask=valid)
                    plsc.store_scatter(posb, [slots], pz, mask=valid)
                    return (nr + jnp.sum(valid.astype(_i32)),)

                return pl.loop(
                    0, (sc + 15) // 16, init_carry=(nrows_in,))(_extract)

            (nrows1,) = pl.loop(
                0, nseg, init_carry=(nrows0,))(_segment)
            return (nrows1,)

        (nrows_end,) = _chunk

        @pl.when(nrows_end > 0)
        def _():
            _flush()

        return (p + 1, tot)

    lax.while_loop(_pass_cond, _pass_body, (_i32(0), _i32(0)))


def kernel(row_offsets, value_tensors, nnz_array, output_shape, table):
    del row_offsets, nnz_array, output_shape
    mesh = plsc.VectorSubcoreMesh(core_axis_name="c", subcore_axis_name="s")
    sweep = pl.kernel(
        _sweep_body,
        out_type=jax.ShapeDtypeStruct((OUT_ROWS, 128), jnp.float32),
        mesh=mesh,
        compiler_params=pltpu.CompilerParams(
            use_tc_tiling_on_sc=True, needs_layout_passes=False),
        scratch_types=[
            pltpu.VMEM((IW,), _i32),
            pltpu.VMEM((HCAP + 16,), _i32),
            pltpu.VMEM((HCAP + 16,), _i32),
            pltpu.VMEM((CSEG + 16,), _i32),
            pltpu.VMEM((CSEG + 16,), _i32),
            pltpu.VMEM((8, 8, CHUNK_COLS), jnp.float32),
            pltpu.VMEM((ROWCAP, 128), jnp.float32),
            pltpu.VMEM((ROWCAP,), _i32),
            pltpu.SemaphoreType.DMA,
            pltpu.SemaphoreType.DMA,
        ],
    )
    out = sweep(table.T, value_tensors)
    return out[:NNZ, :EMB].reshape(B, SLOT, EMB)


# speedup vs baseline: 1.0539x; 1.0539x over previous
import jax
import jax.numpy as jnp
from jax import lax
from jax.experimental import pallas as pl
from jax.experimental.pallas import tpu as pltpu
from jax.experimental.pallas import tpu_sc as plsc

B = 4096
SLOT = 26
EMB = 64
VOCAB = 1000000
NNZ = B * SLOT

NC = 2
NS = 16
NW = NC * NS
WSPAN = 31360
PADCOLS = 1000064
CHUNK_COLS = 1024
IW = 4096
NWIN = NNZ // IW
HCAP = 8192
CSEG = 2048
OUT_ROWS = NNZ + 128

_i32 = jnp.int32


def _sweep_body(tab_t, idx_hbm, out_hbm, idxwin, hitv, hitp, chv, chp,
                stage, rows, posb, isem, dsem, osem):
    wid = lax.axis_index("s") * NC + lax.axis_index("c")
    lo = wid * WSPAN
    hi = jnp.minimum(lo + WSPAN, VOCAB)
    nchunk = (hi - lo + CHUNK_COLS - 1) // CHUNK_COLS
    iota = lax.iota(_i32, 16)
    sink = jnp.full((16,), NNZ, _i32)

    def _reset_posb():
        for g in range(8):
            posb[0, pl.ds(g * 16, 16)] = sink

    def _flush():
        pltpu.async_copy(rows, out_hbm.at[posb.at[0]], osem).wait()
        _reset_posb()

    _reset_posb()

    def _pass_cond(st):
        p, tot = st
        return jnp.logical_or(p == 0, p * HCAP < tot)

    def _pass_body(st):
        p, _ = st
        done = p * HCAP

        pltpu.async_copy(idx_hbm.at[pl.ds(0, IW)],
                         idxwin.at[pl.ds(0, IW)], isem)

        def _filter_window(w, c):
            par = (w % 2) * IW
            pltpu.make_async_copy(idx_hbm.at[pl.ds(w * IW, IW)],
                                  idxwin.at[pl.ds(par, IW)], isem).wait()

            @pl.when(w + 1 < NWIN)
            def _():
                npar = ((w + 1) % 2) * IW
                pltpu.async_copy(idx_hbm.at[pl.ds((w + 1) * IW, IW)],
                                 idxwin.at[pl.ds(npar, IW)], isem)

            def _grp(g, c2):
                tot2, hs2 = c2
                v = idxwin[pl.ds(par + g * 16, 16)]
                m = jnp.logical_and(v >= lo, v < hi)
                mi = m.astype(_i32)
                rank = tot2 + plsc.cumsum(mi)
                keep = jnp.logical_and(
                    m, jnp.logical_and(rank > done, rank <= done + HCAP))
                plsc.store_compressed(hitv.at[pl.ds(hs2, 16)], v, mask=keep)
                plsc.store_compressed(
                    hitp.at[pl.ds(hs2, 16)], w * IW + g * 16 + iota,
                    mask=keep)
                return (tot2 + jnp.sum(mi),
                        hs2 + jnp.sum(keep.astype(_i32)))

            return pl.loop(0, IW // 16, init_carry=c, unroll=8)(_grp)

        tot, hs = pl.loop(
            0, NWIN, init_carry=(_i32(0), _i32(0)))(_filter_window)

        @pl.loop(0, nchunk, init_carry=(_i32(0),))
        def _chunk(c, cc):
            (nr_c,) = cc
            clo = lo + c * CHUNK_COLS
            chi = jnp.minimum(clo + CHUNK_COLS, hi)
            col0 = jnp.minimum(clo, PADCOLS - CHUNK_COLS)
            src = tab_t.at[pl.ds(0, EMB), pl.ds(col0, CHUNK_COLS)]
            pltpu.async_copy(src, stage, dsem)
            pltpu.make_async_copy(src, stage, dsem).wait()

            def _rescan(s):
                def _body(g, c3):
                    rc, sc = c3
                    for k in range(4):
                        off = g * 64 + k * 16
                        v = hitv[pl.ds(off, 16)]
                        pz = hitp[pl.ds(off, 16)]
                        gv = jnp.logical_and(
                            iota < hs - off,
                            jnp.logical_and(v >= clo, v < chi))
                        gi = gv.astype(_i32)
                        rank = rc + plsc.cumsum(gi)
                        keep = jnp.logical_and(
                            gv, jnp.logical_and(rank > s * CSEG,
                                                rank <= s * CSEG + CSEG))
                        plsc.store_compressed(
                            chv.at[pl.ds(sc, 16)], v, mask=keep)
                        plsc.store_compressed(
                            chp.at[pl.ds(sc, 16)], pz, mask=keep)
                        rc = rc + jnp.sum(gi)
                        sc = sc + jnp.sum(keep.astype(_i32))
                    return (rc, sc)

                return pl.loop(0, (hs + 63) // 64,
                               init_carry=(_i32(0), _i32(0)))(_body)

            def _extract_seg(sc, carry):
                def _extract(g, c4):
                    (nr_in,) = c4

                    @pl.when(nr_in > 112)
                    def _():
                        _flush()

                    nr = jnp.where(nr_in > 112, _i32(0), nr_in)
                    zeros = jnp.full((16,), 0, _i32)
                    v = chv[pl.ds(g * 16, 16)]
                    pz = chp[pl.ds(g * 16, 16)]
                    valid = iota < sc - g * 16
                    vrel = v - col0
                    slots = nr + iota
                    for e in range(EMB):
                        colv = plsc.load_gather(
                            stage,
                            [jnp.full((16,), e, _i32), vrel], mask=valid)
                        plsc.store_scatter(
                            rows, [slots, jnp.full((16,), e, _i32)],
                            colv, mask=valid)
                    plsc.store_scatter(posb, [zeros, slots], pz, mask=valid)
                    return (nr + jnp.sum(valid.astype(_i32)),)

                return pl.loop(0, (sc + 15) // 16, init_carry=carry)(_extract)

            rc, sc0 = _rescan(_i32(0))
            out_carry = _extract_seg(sc0, (nr_c,))

            @pl.loop(1, jnp.maximum(_i32(1), (rc + CSEG - 1) // CSEG),
                     init_carry=out_carry)
            def _more_segs(s, c5):
                _, sc_s = _rescan(s)
                return _extract_seg(sc_s, c5)

            return _more_segs

        (nr_end,) = _chunk

        @pl.when(nr_end > 0)
        def _():
            _flush()

        return (p + 1, tot)

    lax.while_loop(_pass_cond, _pass_body, (_i32(0), _i32(0)))


def kernel(row_offsets, value_tensors, nnz_array, output_shape, table):
    del row_offsets, nnz_array, output_shape
    mesh = plsc.VectorSubcoreMesh(core_axis_name="c", subcore_axis_name="s")
    sweep = pl.kernel(
        _sweep_body,
        out_type=jax.ShapeDtypeStruct((OUT_ROWS, 128), jnp.float32),
        mesh=mesh,
        compiler_params=pltpu.CompilerParams(
            use_tc_tiling_on_sc=True, needs_layout_passes=False),
        scratch_types=[
            pltpu.VMEM((2 * IW,), _i32),
            pltpu.VMEM((HCAP + 16,), _i32),
            pltpu.VMEM((HCAP + 16,), _i32),
            pltpu.VMEM((CSEG + 16,), _i32),
            pltpu.VMEM((CSEG + 16,), _i32),
            pltpu.VMEM((EMB, CHUNK_COLS), jnp.float32),
            pltpu.VMEM((128, 128), jnp.float32),
            pltpu.VMEM((1, 128), _i32),
            pltpu.SemaphoreType.DMA,
            pltpu.SemaphoreType.DMA,
            pltpu.SemaphoreType.DMA,
        ],
    )
    out = sweep(table.T, value_tensors)
    return out[:NNZ, :EMB].reshape(B, SLOT, EMB)


# speedup vs baseline: 1.2608x; 1.1963x over previous
import jax
import jax.numpy as jnp
from jax import lax
from jax.experimental import pallas as pl
from jax.experimental.pallas import tpu as pltpu
from jax.experimental.pallas import tpu_sc as plsc

B = 4096
SLOT = 26
EMB = 64
NNZ = B * SLOT

NC = 2
NS = 16
NW = NC * NS
PER_W = NNZ // NW
SUB = 128
NSUB = PER_W // SUB


CHUNK = 13
NCHUNK = NSUB // CHUNK
ROWS_C = CHUNK * SUB


def _gather_body(table_hbm, idx_hbm, out_hbm, idx_v, buf, sem):
    wid = lax.axis_index("s") * NC + lax.axis_index("c")
    base = wid * PER_W

    pltpu.sync_copy(idx_hbm.at[pl.ds(base, PER_W)], idx_v)

    for c in range(NCHUNK):
        for k in range(CHUNK):
            j = c * CHUNK + k
            pltpu.async_copy(
                table_hbm.at[idx_v.at[pl.ds(j * SUB, SUB)]],
                buf.at[pl.ds(k * SUB, SUB)],
                sem,
            )
        pltpu.make_async_copy(
            out_hbm.at[pl.ds(base + c * ROWS_C, ROWS_C)], buf, sem
        ).wait()
        pltpu.sync_copy(buf, out_hbm.at[pl.ds(base + c * ROWS_C, ROWS_C)])


def kernel(row_offsets, value_tensors, nnz_array, output_shape, table):
    del row_offsets, nnz_array, output_shape
    mesh = plsc.VectorSubcoreMesh(core_axis_name="c", subcore_axis_name="s")
    gather = pl.kernel(
        _gather_body,
        out_type=jax.ShapeDtypeStruct((NNZ, EMB), jnp.float32),
        mesh=mesh,
        compiler_params=pltpu.CompilerParams(use_tc_tiling_on_sc=False),
        scratch_types=[
            pltpu.VMEM((PER_W,), jnp.int32),
            pltpu.VMEM((ROWS_C, EMB), jnp.float32),
            pltpu.SemaphoreType.DMA,
        ],
    )
    out = gather(table, value_tensors)
    return out.reshape(B, SLOT, EMB)
